# Initial kernel scaffold; baseline (speedup 1.0000x reference)
import jax
import jax.numpy as jnp
from jax.experimental import pallas as pl


def kernel(inputs, codebook):
    raise NotImplementedError("write your pallas kernel here")


# speedup vs baseline: 1.1750x; 1.1750x over previous
import functools

import jax
import jax.numpy as jnp
from jax import lax
from jax.experimental import pallas as pl
from jax.experimental.pallas import tpu as pltpu
from jax.experimental.pallas import tpu_sc as plsc

NUM_CODES = 1024
DIM = 64
TOK = 32 * 576
BLK = 512
NBLK = TOK // BLK


def _vq_body(x_ref, cb_ref, cbt_ref, idx_ref, vq_ref, perp_ref,
             hist_ref, acc_ref):
    k = pl.program_id(0)

    @pl.when(k == 0)
    def _init():
        hist_ref[...] = jnp.zeros_like(hist_ref)
        acc_ref[0, 0] = 0.0

    x = x_ref[...]
    cb = cb_ref[...]
    cbt = cbt_ref[...]

    s = jnp.dot(x, cbt, preferred_element_type=jnp.float32)
    a = jnp.sum(x * x, axis=1, keepdims=True)
    b = jnp.sum(cb * cb, axis=1)
    dist = (a + b[None, :]) - 2.0 * s

    minval = jnp.min(dist, axis=1, keepdims=True)
    iota = lax.broadcasted_iota(jnp.int32, dist.shape, 1)
    idx = jnp.min(jnp.where(dist == minval, iota, NUM_CODES), axis=1)
    idx_ref[0, 0, :] = idx

    onehot = (iota == idx[:, None]).astype(jnp.float32)
    hist_ref[...] += jnp.sum(onehot, axis=0, keepdims=True)
    acc_ref[0, 0] += jnp.sum(minval)

    @pl.when(k == NBLK - 1)
    def _fini():
        p = hist_ref[0, :] * (1.0 / TOK)
        ent = jnp.sum(p * jnp.log(p + 1e-10))
        perp_ref[...] = jnp.full((1, 1), jnp.exp(-ent), jnp.float32)
        v = acc_ref[0, 0] * (1.0 / (TOK * DIM))
        vq_ref[...] = jnp.full((1, 1), v + 0.25 * v, jnp.float32)


_vq_call = pl.pallas_call(
    _vq_body,
    grid=(NBLK,),
    in_specs=[
        pl.BlockSpec((BLK, DIM), lambda k: (k, 0)),
        pl.BlockSpec((NUM_CODES, DIM), lambda k: (0, 0)),
        pl.BlockSpec((DIM, NUM_CODES), lambda k: (0, 0)),
    ],
    out_specs=[
        pl.BlockSpec((1, 1, BLK), lambda k: (k, 0, 0)),
        pl.BlockSpec((1, 1), lambda k: (0, 0)),
        pl.BlockSpec((1, 1), lambda k: (0, 0)),
    ],
    out_shape=[
        jax.ShapeDtypeStruct((NBLK, 1, BLK), jnp.int32),
        jax.ShapeDtypeStruct((1, 1), jnp.float32),
        jax.ShapeDtypeStruct((1, 1), jnp.float32),
    ],
    scratch_shapes=[
        pltpu.VMEM((1, NUM_CODES), jnp.float32),
        pltpu.SMEM((1, 1), jnp.float32),
    ],
)


_NC = 2
_NS = 16
NW = _NC * _NS
TPW = TOK // NW
CH = 96
NCH = TPW // CH


@functools.cache
def _sc_gather():
    mesh = plsc.VectorSubcoreMesh(core_axis_name="c", subcore_axis_name="s")

    @functools.partial(
        pl.kernel,
        mesh=mesh,
        compiler_params=pltpu.CompilerParams(use_tc_tiling_on_sc=False),
        out_type=jax.ShapeDtypeStruct((TOK, DIM), jnp.float32),
        scratch_types=[
            pltpu.VMEM((NCH, CH), jnp.int32),
            pltpu.VMEM((TPW, DIM), jnp.float32),
            pltpu.SemaphoreType.DMA,
        ],
    )
    def gather(cb_hbm, idx_hbm, out_hbm, idx_v, rows_v, sem):
        w = lax.axis_index("s") * _NC + lax.axis_index("c")
        pltpu.sync_copy(idx_hbm.at[w], idx_v)
        for j in range(NCH):
            pltpu.async_copy(cb_hbm.at[idx_v.at[j]],
                             rows_v.at[pl.ds(j * CH, CH)], sem).wait()
        pltpu.sync_copy(rows_v, out_hbm.at[pl.ds(w * TPW, TPW)])

    return gather


def kernel(inputs, codebook):
    x = inputs.reshape(TOK, DIM)
    idx3, vq, perp = _vq_call(x, codebook, codebook.T)
    idx_flat = idx3.reshape(TOK)
    quantized = _sc_gather()(codebook, idx_flat.reshape(NW, NCH, CH))
    return (
        quantized.reshape(inputs.shape),
        idx_flat.reshape(inputs.shape[0], inputs.shape[1]),
        vq[0, 0],
        perp[0, 0],
    )


# speedup vs baseline: 1.3453x; 1.1449x over previous
import functools

import jax
import jax.numpy as jnp
from jax import lax
from jax.experimental import pallas as pl
from jax.experimental.pallas import tpu as pltpu
from jax.experimental.pallas import tpu_sc as plsc

NUM_CODES = 1024
DIM = 64
TOK = 32 * 576
BLK = 1024
NBLK = TOK // BLK


def _vq_body(x_ref, cb_ref, cbt2_ref, idx_ref, vq_ref, perp_ref,
             hist_ref, b_ref, iotaf_ref, acc_ref):
    k = pl.program_id(0)

    @pl.when(k == 0)
    def _init():
        hist_ref[...] = jnp.zeros_like(hist_ref)
        cb = cb_ref[...]
        b_ref[...] = jnp.sum(cb * cb, axis=1)[None, :]
        iotaf_ref[...] = lax.broadcasted_iota(
            jnp.int32, (1, NUM_CODES), 1).astype(jnp.float32)
        acc_ref[0, 0] = 0.0

    x = x_ref[...]
    s2 = jnp.dot(x, cbt2_ref[...], preferred_element_type=jnp.float32)
    a = jnp.sum(x * x, axis=1, keepdims=True)
    dist = (a + b_ref[...]) + s2

    minval = jnp.min(dist, axis=1, keepdims=True)
    iota = jnp.broadcast_to(iotaf_ref[...], dist.shape)
    idxf = jnp.min(jnp.where(dist == minval, iota, 65536.0), axis=1)
    idx_ref[...] = idxf.astype(jnp.int32)

    onehot = (iota == idxf[:, None]).astype(jnp.float32)
    hist_ref[...] += jnp.sum(onehot, axis=0, keepdims=True)
    acc_ref[0, 0] += jnp.sum(minval)

    @pl.when(k == NBLK - 1)
    def _fini():
        p = hist_ref[0, :] * (1.0 / TOK)
        ent = jnp.sum(p * jnp.log(p + 1e-10))
        perp_ref[...] = jnp.full((1, 1), jnp.exp(-ent), jnp.float32)
        v = acc_ref[0, 0] * (1.0 / (TOK * DIM))
        vq_ref[...] = jnp.full((1, 1), v + 0.25 * v, jnp.float32)


_vq_call = pl.pallas_call(
    _vq_body,
    grid=(NBLK,),
    in_specs=[
        pl.BlockSpec((BLK, DIM), lambda k: (k, 0)),
        pl.BlockSpec((NUM_CODES, DIM), lambda k: (0, 0)),
        pl.BlockSpec((DIM, NUM_CODES), lambda k: (0, 0)),
    ],
    out_specs=[
        pl.BlockSpec((BLK,), lambda k: (k,)),
        pl.BlockSpec((1, 1), lambda k: (0, 0)),
        pl.BlockSpec((1, 1), lambda k: (0, 0)),
    ],
    out_shape=[
        jax.ShapeDtypeStruct((TOK,), jnp.int32),
        jax.ShapeDtypeStruct((1, 1), jnp.float32),
        jax.ShapeDtypeStruct((1, 1), jnp.float32),
    ],
    scratch_shapes=[
        pltpu.VMEM((1, NUM_CODES), jnp.float32),
        pltpu.VMEM((1, NUM_CODES), jnp.float32),
        pltpu.VMEM((1, NUM_CODES), jnp.float32),
        pltpu.SMEM((1, 1), jnp.float32),
    ],
)


_NC = 2
_NS = 16
NW = _NC * _NS
TPW = TOK // NW
CH = 96
NCH = TPW // CH


@functools.cache
def _sc_gather():
    mesh = plsc.VectorSubcoreMesh(core_axis_name="c", subcore_axis_name="s")

    @functools.partial(
        pl.kernel,
        mesh=mesh,
        compiler_params=pltpu.CompilerParams(use_tc_tiling_on_sc=False),
        out_type=jax.ShapeDtypeStruct((TOK, DIM), jnp.float32),
        scratch_types=[
            pltpu.VMEM((NCH, CH), jnp.int32),
            pltpu.VMEM((TPW, DIM), jnp.float32),
            pltpu.SemaphoreType.DMA,
        ],
    )
    def gather(cb_hbm, idx_hbm, out_hbm, idx_v, rows_v, sem):
        w = lax.axis_index("s") * _NC + lax.axis_index("c")
        pltpu.sync_copy(idx_hbm.at[w], idx_v)
        for j in range(NCH):
            pltpu.async_copy(cb_hbm.at[idx_v.at[j]],
                             rows_v.at[pl.ds(j * CH, CH)], sem).wait()
        pltpu.sync_copy(rows_v, out_hbm.at[pl.ds(w * TPW, TPW)])

    return gather


def kernel(inputs, codebook):
    x = inputs.reshape(TOK, DIM)
    cbt2 = -2.0 * codebook.T
    idx_flat, vq, perp = _vq_call(x, codebook, cbt2)
    quantized = _sc_gather()(codebook, idx_flat.reshape(NW, NCH, CH))
    return (
        quantized.reshape(inputs.shape),
        idx_flat.reshape(inputs.shape[0], inputs.shape[1]),
        vq[0, 0],
        perp[0, 0],
    )


# speedup vs baseline: 1.3693x; 1.0179x over previous
import functools

import jax
import jax.numpy as jnp
from jax import lax
from jax.experimental import pallas as pl
from jax.experimental.pallas import tpu as pltpu
from jax.experimental.pallas import tpu_sc as plsc

NUM_CODES = 1024
DIM = 64
BATCH = 32
SEQ = 576
TOK = BATCH * SEQ
ROWS = 4
BLK = ROWS * SEQ
NBLK = BATCH // ROWS


def _vq_body(x_ref, cb_ref, idx_ref, vq_ref, perp_ref,
             hist_ref, b_ref, iotaf_ref, cb2_ref, acc_ref):
    k = pl.program_id(0)

    @pl.when(k == 0)
    def _init():
        hist_ref[...] = jnp.zeros_like(hist_ref)
        cb = cb_ref[...]
        b_ref[...] = jnp.sum(cb * cb, axis=1)[None, :]
        cb2_ref[...] = -2.0 * cb
        iotaf_ref[...] = lax.broadcasted_iota(
            jnp.int32, (1, NUM_CODES), 1).astype(jnp.float32)
        acc_ref[0, 0] = 0.0

    x = x_ref[...].reshape(BLK, DIM)
    s2 = lax.dot_general(x, cb2_ref[...], (((1,), (1,)), ((), ())),
                         preferred_element_type=jnp.float32)
    a = jnp.sum(x * x, axis=1, keepdims=True)
    dist = (a + b_ref[...]) + s2

    minval = jnp.min(dist, axis=1, keepdims=True)
    iota = jnp.broadcast_to(iotaf_ref[...], dist.shape)
    idxf = jnp.min(jnp.where(dist == minval, iota, 65536.0), axis=1)
    idx_ref[pl.ds(k * BLK, BLK)] = idxf.astype(jnp.int32)

    onehot = (iota == idxf[:, None]).astype(jnp.float32)
    hist_ref[...] += jnp.sum(onehot, axis=0, keepdims=True)
    acc_ref[0, 0] += jnp.sum(minval)

    @pl.when(k == NBLK - 1)
    def _fini():
        p = hist_ref[0, :] * (1.0 / TOK)
        ent = jnp.sum(p * jnp.log(p + 1e-10))
        perp_ref[...] = jnp.full((1, 1), jnp.exp(-ent), jnp.float32)
        v = acc_ref[0, 0] * (1.0 / (TOK * DIM))
        vq_ref[...] = jnp.full((1, 1), v + 0.25 * v, jnp.float32)


_vq_call = pl.pallas_call(
    _vq_body,
    grid=(NBLK,),
    in_specs=[
        pl.BlockSpec((ROWS, SEQ, DIM), lambda k: (k, 0, 0)),
        pl.BlockSpec((NUM_CODES, DIM), lambda k: (0, 0)),
    ],
    out_specs=[
        pl.BlockSpec((TOK,), lambda k: (0,)),
        pl.BlockSpec((1, 1), lambda k: (0, 0)),
        pl.BlockSpec((1, 1), lambda k: (0, 0)),
    ],
    out_shape=[
        jax.ShapeDtypeStruct((TOK,), jnp.int32),
        jax.ShapeDtypeStruct((1, 1), jnp.float32),
        jax.ShapeDtypeStruct((1, 1), jnp.float32),
    ],
    scratch_shapes=[
        pltpu.VMEM((1, NUM_CODES), jnp.float32),
        pltpu.VMEM((1, NUM_CODES), jnp.float32),
        pltpu.VMEM((1, NUM_CODES), jnp.float32),
        pltpu.VMEM((NUM_CODES, DIM), jnp.float32),
        pltpu.SMEM((1, 1), jnp.float32),
    ],
)


_NC = 2
_NS = 16
NW = _NC * _NS
TPW = TOK // NW
CH = 96
NCH = TPW // CH


@functools.cache
def _sc_gather():
    mesh = plsc.VectorSubcoreMesh(core_axis_name="c", subcore_axis_name="s")

    @functools.partial(
        pl.kernel,
        mesh=mesh,
        compiler_params=pltpu.CompilerParams(use_tc_tiling_on_sc=False),
        out_type=jax.ShapeDtypeStruct((TOK, DIM), jnp.float32),
        scratch_types=[
            pltpu.VMEM((TPW,), jnp.int32),
            pltpu.VMEM((TPW, DIM), jnp.float32),
            pltpu.SemaphoreType.DMA,
        ],
    )
    def gather(cb_hbm, idx_hbm, out_hbm, idx_v, rows_v, sem):
        w = lax.axis_index("s") * _NC + lax.axis_index("c")
        pltpu.sync_copy(idx_hbm.at[pl.ds(w * TPW, TPW)], idx_v)
        for j in range(NCH):
            pltpu.async_copy(cb_hbm.at[idx_v.at[pl.ds(j * CH, CH)]],
                             rows_v.at[pl.ds(j * CH, CH)], sem).wait()
        pltpu.sync_copy(rows_v, out_hbm.at[pl.ds(w * TPW, TPW)])

    return gather


def kernel(inputs, codebook):
    idx_flat, vq, perp = _vq_call(inputs, codebook)
    quantized = _sc_gather()(codebook, idx_flat)
    return (
        quantized.reshape(inputs.shape),
        idx_flat.reshape(BATCH, SEQ),
        vq[0, 0],
        perp[0, 0],
    )


# speedup vs baseline: 1.5001x; 1.0955x over previous
import functools

import jax
import jax.numpy as jnp
from jax import lax
from jax.experimental import pallas as pl
from jax.experimental.pallas import tpu as pltpu
from jax.experimental.pallas import tpu_sc as plsc

NUM_CODES = 1024
DIM = 64
BATCH = 32
SEQ = 576
TOK = BATCH * SEQ
ROWS = 4
BLK = ROWS * SEQ
NBLK = BATCH // ROWS


def _vq_body(x_ref, cb_ref, idx_ref, vq_ref, perp_ref,
             hist_ref, b_ref, iotaf_ref, cb2_ref, acc_ref):
    k = pl.program_id(0)

    @pl.when(k == 0)
    def _init():
        hist_ref[...] = jnp.zeros_like(hist_ref)
        cb = cb_ref[...]
        b_ref[...] = jnp.sum(cb * cb, axis=1)[None, :]
        cb2_ref[...] = -2.0 * cb
        iotaf_ref[...] = lax.broadcasted_iota(
            jnp.int32, (1, NUM_CODES), 1).astype(jnp.float32)
        acc_ref[0, 0] = 0.0

    x = x_ref[...].reshape(BLK, DIM)
    s2 = lax.dot_general(x, cb2_ref[...], (((1,), (1,)), ((), ())),
                         preferred_element_type=jnp.float32)
    a = jnp.sum(x * x, axis=1, keepdims=True)
    dist = (a + b_ref[...]) + s2

    minval = jnp.min(dist, axis=1, keepdims=True)
    iota = jnp.broadcast_to(iotaf_ref[...], dist.shape)
    idxf = jnp.min(jnp.where(dist == minval, iota, 65536.0), axis=1)
    idx_ref[pl.ds(k * BLK, BLK)] = idxf.astype(jnp.int32)

    onehot = (iota == idxf[:, None]).astype(jnp.float32)
    ones_row = jnp.ones((1, BLK), jnp.float32)
    hist_ref[...] += jnp.dot(ones_row, onehot,
                             preferred_element_type=jnp.float32)
    acc_ref[0, 0] += jnp.sum(minval)

    @pl.when(k == NBLK - 1)
    def _fini():
        p = hist_ref[0, :] * (1.0 / TOK)
        ent = jnp.sum(p * jnp.log(p + 1e-10))
        perp_ref[...] = jnp.full((1, 1), jnp.exp(-ent), jnp.float32)
        v = acc_ref[0, 0] * (1.0 / (TOK * DIM))
        vq_ref[...] = jnp.full((1, 1), v + 0.25 * v, jnp.float32)


_vq_call = pl.pallas_call(
    _vq_body,
    grid=(NBLK,),
    in_specs=[
        pl.BlockSpec((ROWS, SEQ, DIM), lambda k: (k, 0, 0)),
        pl.BlockSpec((NUM_CODES, DIM), lambda k: (0, 0)),
    ],
    out_specs=[
        pl.BlockSpec((TOK,), lambda k: (0,)),
        pl.BlockSpec((1, 1), lambda k: (0, 0)),
        pl.BlockSpec((1, 1), lambda k: (0, 0)),
    ],
    out_shape=[
        jax.ShapeDtypeStruct((TOK,), jnp.int32),
        jax.ShapeDtypeStruct((1, 1), jnp.float32),
        jax.ShapeDtypeStruct((1, 1), jnp.float32),
    ],
    scratch_shapes=[
        pltpu.VMEM((1, NUM_CODES), jnp.float32),
        pltpu.VMEM((1, NUM_CODES), jnp.float32),
        pltpu.VMEM((1, NUM_CODES), jnp.float32),
        pltpu.VMEM((NUM_CODES, DIM), jnp.float32),
        pltpu.SMEM((1, 1), jnp.float32),
    ],
)


_NC = 2
_NS = 16
NW = _NC * _NS
TPW = TOK // NW
CH = 96
NCH = TPW // CH


@functools.cache
def _sc_gather():
    mesh = plsc.VectorSubcoreMesh(core_axis_name="c", subcore_axis_name="s")

    @functools.partial(
        pl.kernel,
        mesh=mesh,
        compiler_params=pltpu.CompilerParams(use_tc_tiling_on_sc=False),
        out_type=jax.ShapeDtypeStruct((BATCH, SEQ, DIM), jnp.float32),
        scratch_types=[
            pltpu.VMEM((TPW,), jnp.int32),
            pltpu.VMEM((TPW, DIM), jnp.float32),
            pltpu.SemaphoreType.DMA,
        ],
    )
    def gather(cb_hbm, idx_hbm, out_hbm, idx_v, rows_v, sem):
        w = lax.axis_index("s") * _NC + lax.axis_index("c")
        pltpu.sync_copy(idx_hbm.at[pl.ds(w * TPW, TPW)], idx_v)
        for j in range(NCH):
            pltpu.async_copy(cb_hbm.at[idx_v.at[pl.ds(j * CH, CH)]],
                             rows_v.at[pl.ds(j * CH, CH)], sem).wait()
        pltpu.sync_copy(rows_v, out_hbm.at[w])

    return gather


def kernel(inputs, codebook):
    idx_flat, vq, perp = _vq_call(inputs, codebook)
    quantized = _sc_gather()(codebook, idx_flat)
    return (
        quantized,
        idx_flat.reshape(BATCH, SEQ),
        vq[0, 0],
        perp[0, 0],
    )
